# baseline (speedup 1.0000x reference)
import jax
import jax.numpy as jnp
from jax import lax
from jax.experimental import pallas as pl
from jax.experimental.pallas import tpu as pltpu
from jax.experimental.pallas import tpu_sc as plsc

NC, NS, L = 2, 16, 16
NW = NC * NS
B = 16384
F = 32
N = 1000000
BPW = B // NW
CHUNK = 128
NCH = BPW // CHUNK


def _sc_body(user_hbm, item_hbm, ut_hbm, it_hbm, out_hbm,
             uidx_v, iidx_v, gidx_v, ug_v, ig_v, out_v, sem):
    wid = lax.axis_index("s") * NC + lax.axis_index("c")
    base = wid * BPW

    pltpu.sync_copy(user_hbm.at[pl.ds(base, BPW)], uidx_v)
    pltpu.sync_copy(item_hbm.at[pl.ds(base, BPW)], iidx_v)

    def build(t, idx_v):
        def fac(c, carry):
            def vec(i, carry2):
                r = idx_v[pl.ds(i * L, L)]
                gidx_v[t, c, pl.ds(i * L, L)] = r + c * N
                return carry2
            return lax.fori_loop(0, BPW // L, vec, carry)
        lax.fori_loop(0, F, fac, 0)

    build(0, uidx_v)
    build(1, iidx_v)

    copies = []
    for c in range(F):
        for k in range(NCH):
            sl = pl.ds(k * CHUNK, CHUNK)
            copies.append(pltpu.async_copy(
                ut_hbm.at[gidx_v.at[0, c, sl]], ug_v.at[c, sl], sem))
            copies.append(pltpu.async_copy(
                it_hbm.at[gidx_v.at[1, c, sl]], ig_v.at[c, sl], sem))
    for cp in copies:
        cp.wait()

    def group(g, carry):
        b0 = g * L
        acc = jnp.zeros((L,), jnp.float32)
        for c in range(F):
            acc = acc + ug_v[c, pl.ds(b0, L)] * ig_v[c, pl.ds(b0, L)]
        out_v[pl.ds(b0, L)] = acc
        return carry

    lax.fori_loop(0, BPW // L, group, 0)

    pltpu.sync_copy(out_v, out_hbm.at[pl.ds(base, BPW)])


def kernel(user, item, user_table, item_table):
    uflat = user_table.T.reshape(-1)
    iflat = item_table.T.reshape(-1)
    mesh = plsc.VectorSubcoreMesh(core_axis_name="c", subcore_axis_name="s")
    k = pl.kernel(
        _sc_body,
        out_type=jax.ShapeDtypeStruct((B,), jnp.float32),
        mesh=mesh,
        compiler_params=pltpu.CompilerParams(
            needs_layout_passes=False, use_tc_tiling_on_sc=False),
        scratch_types=[
            pltpu.VMEM((BPW,), jnp.int32),
            pltpu.VMEM((BPW,), jnp.int32),
            pltpu.VMEM((2, F, BPW), jnp.int32),
            pltpu.VMEM((F, BPW), jnp.float32),
            pltpu.VMEM((F, BPW), jnp.float32),
            pltpu.VMEM((BPW,), jnp.float32),
            pltpu.SemaphoreType.DMA,
        ],
    )
    return k(user, item, uflat, iflat)


# speedup vs baseline: 5.5936x; 5.5936x over previous
import jax
import jax.numpy as jnp
from jax import lax
from jax.experimental import pallas as pl
from jax.experimental.pallas import tpu as pltpu
from jax.experimental.pallas import tpu_sc as plsc

NC, NS, L = 2, 16, 16
NW = NC * NS
B = 16384
F = 32
N = 1000000
G = 4
BPW = B // NW
CHUNK = 128
NCH = BPW // CHUNK


def _sc_body(user_hbm, item_hbm, ut_hbm, it_hbm, out_hbm,
             uidx_v, iidx_v, uslab_v, islab_v, urows_v, irows_v, out_v, sem):
    wid = lax.axis_index("s") * NC + lax.axis_index("c")
    base = wid * BPW

    pltpu.sync_copy(user_hbm.at[pl.ds(base, BPW)], uidx_v)
    pltpu.sync_copy(item_hbm.at[pl.ds(base, BPW)], iidx_v)

    def slabify(i, carry):
        sl = pl.ds(i * L, L)
        uslab_v[sl] = jax.lax.shift_right_logical(uidx_v[sl], 2)
        islab_v[sl] = jax.lax.shift_right_logical(iidx_v[sl], 2)
        return carry
    lax.fori_loop(0, BPW // L, slabify, 0)

    lanes = lax.iota(jnp.int32, L)

    def chunk(k, carry):
        csl = pl.ds(k * CHUNK, CHUNK)
        cp_u = pltpu.async_copy(ut_hbm.at[uslab_v.at[csl]], urows_v, sem)
        cp_i = pltpu.async_copy(it_hbm.at[islab_v.at[csl]], irows_v, sem)
        cp_u.wait()
        cp_i.wait()

        def group(g, carry2):
            b0 = k * CHUNK + g * L
            rows = g * L + lanes
            uband = (uidx_v[pl.ds(b0, L)] & 3) * F
            iband = (iidx_v[pl.ds(b0, L)] & 3) * F
            acc = jnp.zeros((L,), jnp.float32)
            for c in range(F):
                u = plsc.load_gather(urows_v, [rows, uband + c])
                v = plsc.load_gather(irows_v, [rows, iband + c])
                acc = acc + u * v
            out_v[pl.ds(b0, L)] = acc
            return carry2

        return lax.fori_loop(0, CHUNK // L, group, carry)

    lax.fori_loop(0, NCH, chunk, 0)

    pltpu.sync_copy(out_v, out_hbm.at[pl.ds(base, BPW)])


def kernel(user, item, user_table, item_table):
    u4 = user_table.reshape(N // G, G * F)
    i4 = item_table.reshape(N // G, G * F)
    mesh = plsc.VectorSubcoreMesh(core_axis_name="c", subcore_axis_name="s")
    k = pl.kernel(
        _sc_body,
        out_type=jax.ShapeDtypeStruct((B,), jnp.float32),
        mesh=mesh,
        compiler_params=pltpu.CompilerParams(
            needs_layout_passes=False, use_tc_tiling_on_sc=True),
        scratch_types=[
            pltpu.VMEM((BPW,), jnp.int32),
            pltpu.VMEM((BPW,), jnp.int32),
            pltpu.VMEM((BPW,), jnp.int32),
            pltpu.VMEM((BPW,), jnp.int32),
            pltpu.VMEM((CHUNK, G * F), jnp.float32),
            pltpu.VMEM((CHUNK, G * F), jnp.float32),
            pltpu.VMEM((BPW,), jnp.float32),
            pltpu.SemaphoreType.DMA,
        ],
    )
    return k(user, item, u4, i4)


# speedup vs baseline: 20.1478x; 3.6020x over previous
import jax
import jax.numpy as jnp
from jax import lax
from jax.experimental import pallas as pl
from jax.experimental.pallas import tpu as pltpu
from jax.experimental.pallas import tpu_sc as plsc

NC, NS, L = 2, 16, 16
NW = NC * NS
B = 16384
F = 32
N = 1000000
NMAIN = 999936
BPW = B // NW
CHUNK = 128
NCH = BPW // CHUNK


W = 35712
NWIN = NMAIN // W


def _relayout_body(ut_hbm, it_hbm, utail_hbm, itail_hbm, ul_hbm, il_hbm,
                   buf0, buf1, sem):
    wid = lax.axis_index("s") * NC + lax.axis_index("c")
    q = wid // 8
    k = wid % 8
    NT = N - NMAIN
    for src, tail, dst in ((ut_hbm, utail_hbm, ul_hbm),
                           (it_hbm, itail_hbm, il_hbm)):
        bufs = (buf0, buf1)
        loads = [None, None]
        loads[0] = pltpu.async_copy(src.at[q, k, pl.ds(0, W)], buf0, sem)
        for j in range(NWIN):
            nxt = (j + 1) % 2
            if j + 1 < NWIN:
                loads[nxt] = pltpu.async_copy(
                    src.at[q, k, pl.ds((j + 1) * W, W)], bufs[nxt], sem)
            loads[j % 2].wait()
            pltpu.sync_copy(bufs[j % 2], dst.at[pl.ds(wid * N + j * W, W)])
        pltpu.sync_copy(tail.at[pl.ds(wid * NT, NT)],
                        buf0.at[pl.ds(0, NT)])
        pltpu.sync_copy(buf0.at[pl.ds(0, NT)],
                        dst.at[pl.ds(wid * N + NMAIN, NT)])


def _gather_body(user_hbm, item_hbm, ul_hbm, il_hbm, out_hbm,
                 uidx_v, iidx_v, gidx_v, ug_v, ig_v, out_v, sem):
    wid = lax.axis_index("s") * NC + lax.axis_index("c")
    base = wid * BPW

    pltpu.sync_copy(user_hbm.at[pl.ds(base, BPW)], uidx_v)
    pltpu.sync_copy(item_hbm.at[pl.ds(base, BPW)], iidx_v)

    def build(t, idx_v):
        def fac(c, carry):
            def vec(i, carry2):
                r = idx_v[pl.ds(i * L, L)]
                gidx_v[t, c, pl.ds(i * L, L)] = r + c * N
                return carry2
            return lax.fori_loop(0, BPW // L, vec, carry)
        lax.fori_loop(0, F, fac, 0)

    build(0, uidx_v)
    build(1, iidx_v)

    copies = []
    for c in range(F):
        for k in range(NCH):
            sl = pl.ds(k * CHUNK, CHUNK)
            copies.append(pltpu.async_copy(
                ul_hbm.at[gidx_v.at[0, c, sl]], ug_v.at[c, sl], sem))
            copies.append(pltpu.async_copy(
                il_hbm.at[gidx_v.at[1, c, sl]], ig_v.at[c, sl], sem))
    for cp in copies:
        cp.wait()

    def group(g, carry):
        b0 = g * L
        acc = jnp.zeros((L,), jnp.float32)
        for c in range(F):
            acc = acc + ug_v[c, pl.ds(b0, L)] * ig_v[c, pl.ds(b0, L)]
        out_v[pl.ds(b0, L)] = acc
        return carry

    lax.fori_loop(0, BPW // L, group, 0)

    pltpu.sync_copy(out_v, out_hbm.at[pl.ds(base, BPW)])


def kernel(user, item, user_table, item_table):
    u3 = user_table.T.reshape(F // 8, 8, N)
    i3 = item_table.T.reshape(F // 8, 8, N)
    utail = user_table[NMAIN:].T.reshape(-1)
    itail = item_table[NMAIN:].T.reshape(-1)
    mesh = plsc.VectorSubcoreMesh(core_axis_name="c", subcore_axis_name="s")

    relayout = pl.kernel(
        _relayout_body,
        out_type=(jax.ShapeDtypeStruct((F * N,), jnp.float32),
                  jax.ShapeDtypeStruct((F * N,), jnp.float32)),
        mesh=mesh,
        compiler_params=pltpu.CompilerParams(
            needs_layout_passes=False, use_tc_tiling_on_sc=True),
        scratch_types=[
            pltpu.VMEM((W,), jnp.float32),
            pltpu.VMEM((W,), jnp.float32),
            pltpu.SemaphoreType.DMA,
        ],
    )
    ul, il = relayout(u3, i3, utail, itail)

    gather = pl.kernel(
        _gather_body,
        out_type=jax.ShapeDtypeStruct((B,), jnp.float32),
        mesh=mesh,
        compiler_params=pltpu.CompilerParams(
            needs_layout_passes=False, use_tc_tiling_on_sc=False),
        scratch_types=[
            pltpu.VMEM((BPW,), jnp.int32),
            pltpu.VMEM((BPW,), jnp.int32),
            pltpu.VMEM((2, F, BPW), jnp.int32),
            pltpu.VMEM((F, BPW), jnp.float32),
            pltpu.VMEM((F, BPW), jnp.float32),
            pltpu.VMEM((BPW,), jnp.float32),
            pltpu.SemaphoreType.DMA,
        ],
    )
    return gather(user, item, ul, il)
